# baseline (speedup 1.0000x reference)
import functools

import jax
import jax.numpy as jnp
from jax import lax
from jax.experimental import pallas as pl
from jax.experimental.pallas import tpu as pltpu
from jax.experimental.pallas import tpu_sc as plsc

_NC, _NS = 2, 16
_NW = _NC * _NS
_N, _B = 4096, 4
_NIDX = _N * _B
_D = 1024
_BPW = _NIDX // _NW
_CH = 32
_NCHUNK = _BPW // _CH
_GRP = _CH // _B
_NBUF = 3


@functools.partial(
    pl.kernel,
    mesh=plsc.VectorSubcoreMesh(core_axis_name="c", subcore_axis_name="s"),
    out_type=jax.ShapeDtypeStruct((_N, _B, _D), jnp.float32),
    scratch_types=[
        pltpu.VMEM((_BPW,), jnp.int32),
        pltpu.VMEM((_CH, _D), jnp.float32),
        pltpu.VMEM((_CH, _D), jnp.float32),
        pltpu.VMEM((_CH, _D), jnp.float32),
        pltpu.SemaphoreType.DMA,
        pltpu.SemaphoreType.DMA,
        pltpu.SemaphoreType.DMA,
        pltpu.SemaphoreType.DMA,
        pltpu.SemaphoreType.DMA,
        pltpu.SemaphoreType.DMA,
    ],
)
def _gather_rows(idx_hbm, table_hbm, out_hbm, idx_v, buf0, buf1, buf2,
                 gsem0, gsem1, gsem2, osem0, osem1, osem2):
    wid = lax.axis_index("s") * _NC + lax.axis_index("c")
    base = wid * _BPW
    nbase = base // _B
    pltpu.sync_copy(idx_hbm.at[pl.ds(base, _BPW)], idx_v)

    bufs = (buf0, buf1, buf2)
    gsems = (gsem0, gsem1, gsem2)
    osems = (osem0, osem1, osem2)

    gathers = [None] * _NCHUNK
    outs = [[] for _ in range(_NBUF)]

    for p in range(min(2, _NCHUNK)):
        gathers[p] = pltpu.async_copy(
            table_hbm.at[idx_v.at[pl.ds(p * _CH, _CH)]], bufs[p], gsems[p])
    for c in range(_NCHUNK):
        nxt = (c + 2) % _NBUF
        if c + 2 < _NCHUNK:
            for o in outs[nxt]:
                o.wait()
            outs[nxt] = []
            gathers[c + 2] = pltpu.async_copy(
                table_hbm.at[idx_v.at[pl.ds((c + 2) * _CH, _CH)]],
                bufs[nxt], gsems[nxt])
        cur = c % _NBUF
        gathers[c].wait()
        for g in range(_GRP):
            outs[cur].append(pltpu.async_copy(
                bufs[cur].at[pl.ds(g * _B, _B)],
                out_hbm.at[nbase + c * _GRP + g],
                osems[cur]))

    for olist in outs:
        for o in olist:
            o.wait()


def kernel(i, pe):
    n, b, d = i.shape
    cm = jnp.zeros((n, n, b), dtype=bool)
    idx = i.reshape(-1)
    pe_g = _gather_rows(idx, pe)
    return (pe_g, cm)


# speedup vs baseline: 1.0288x; 1.0288x over previous
import functools

import jax
import jax.numpy as jnp
from jax import lax
from jax.experimental import pallas as pl
from jax.experimental.pallas import tpu as pltpu
from jax.experimental.pallas import tpu_sc as plsc

_NC, _NS = 2, 16
_NW = _NC * _NS
_N, _B = 4096, 4
_NIDX = _N * _B
_D = 1024
_BPW = _NIDX // _NW
_CH = 32
_NCHUNK = _BPW // _CH
_GRP = _CH // _B
_NBUF = 3


@functools.partial(
    pl.kernel,
    mesh=plsc.VectorSubcoreMesh(core_axis_name="c", subcore_axis_name="s"),
    out_type=jax.ShapeDtypeStruct((_N, _B, _D), jnp.float32),
    scratch_types=[
        pltpu.VMEM((_BPW,), jnp.int32),
        pltpu.VMEM((_CH, _D), jnp.float32),
        pltpu.VMEM((_CH, _D), jnp.float32),
        pltpu.VMEM((_CH, _D), jnp.float32),
        pltpu.SemaphoreType.DMA,
        pltpu.SemaphoreType.DMA,
        pltpu.SemaphoreType.DMA,
        pltpu.SemaphoreType.DMA,
        pltpu.SemaphoreType.DMA,
        pltpu.SemaphoreType.DMA,
    ],
)
def _gather_rows(idx_hbm, table_hbm, out_hbm, idx_v, buf0, buf1, buf2,
                 gsem0, gsem1, gsem2, osem0, osem1, osem2):
    wid = lax.axis_index("s") * _NC + lax.axis_index("c")
    base = wid * _BPW
    out_rows = out_hbm.reshape(_NIDX, _D)
    pltpu.sync_copy(idx_hbm.at[pl.ds(base, _BPW)], idx_v)

    bufs = (buf0, buf1, buf2)
    gsems = (gsem0, gsem1, gsem2)
    osems = (osem0, osem1, osem2)

    gathers = [None] * _NCHUNK
    outs = [[] for _ in range(_NBUF)]

    for p in range(min(2, _NCHUNK)):
        gathers[p] = pltpu.async_copy(
            table_hbm.at[idx_v.at[pl.ds(p * _CH, _CH)]], bufs[p], gsems[p])
    for c in range(_NCHUNK):
        nxt = (c + 2) % _NBUF
        if c + 2 < _NCHUNK:
            for o in outs[nxt]:
                o.wait()
            outs[nxt] = []
            gathers[c + 2] = pltpu.async_copy(
                table_hbm.at[idx_v.at[pl.ds((c + 2) * _CH, _CH)]],
                bufs[nxt], gsems[nxt])
        cur = c % _NBUF
        gathers[c].wait()
        outs[cur].append(pltpu.async_copy(
            bufs[cur], out_rows.at[pl.ds(base + c * _CH, _CH)],
            osems[cur]))

    for olist in outs:
        for o in olist:
            o.wait()


def kernel(i, pe):
    n, b, d = i.shape
    cm = jnp.zeros((n, n, b), dtype=bool)
    idx = i.reshape(-1)
    pe_g = _gather_rows(idx, pe)
    return (pe_g, cm)
